# Initial kernel scaffold; baseline (speedup 1.0000x reference)
import jax
import jax.numpy as jnp
from jax.experimental import pallas as pl


def kernel(token_ids, weights):
    raise NotImplementedError("write your pallas kernel here")


# speedup vs baseline: 1.0224x; 1.0224x over previous
import functools

import jax
import jax.numpy as jnp
from jax import lax
from jax.experimental import pallas as pl
from jax.experimental.pallas import tpu as pltpu
from jax.experimental.pallas import tpu_sc as plsc

D = 32
NC, NS = 2, 16
NW = NC * NS
B = 16384 * 50
CHUNK = 128
ROWS_PER_W = B // (NW * CHUNK)


def _make_kernel():
    mesh = plsc.VectorSubcoreMesh(core_axis_name="c", subcore_axis_name="s")

    @functools.partial(
        pl.kernel,
        mesh=mesh,
        out_type=jax.ShapeDtypeStruct((B, D), jnp.float32),
        scratch_types=[
            pltpu.VMEM((ROWS_PER_W, CHUNK), jnp.int32),
            pltpu.VMEM((CHUNK, D), jnp.float32),
            pltpu.SemaphoreType.DMA,
        ],
        compiler_params=pltpu.CompilerParams(use_tc_tiling_on_sc=False),
    )
    def emb_kernel(idx_hbm, table_hbm, out_hbm, idx_v, rows_v, sem):
        wid = lax.axis_index("s") * NC + lax.axis_index("c")
        row0 = wid * ROWS_PER_W
        base = row0 * CHUNK
        pltpu.sync_copy(idx_hbm.at[pl.ds(row0, ROWS_PER_W)], idx_v)

        def body(r, carry):
            pltpu.async_copy(table_hbm.at[idx_v.at[r]], rows_v, sem).wait()
            pltpu.sync_copy(rows_v, out_hbm.at[pl.ds(base + r * CHUNK, CHUNK)])
            return carry

        lax.fori_loop(0, ROWS_PER_W, body, 0)

    return emb_kernel


_emb = _make_kernel()


@jax.jit
def kernel(token_ids, weights):
    idx = token_ids.reshape(B // CHUNK, CHUNK).astype(jnp.int32)
    out = _emb(idx, weights)
    return out.reshape(token_ids.shape[0], token_ids.shape[1], D)


# speedup vs baseline: 1.1127x; 1.0883x over previous
import functools

import jax
import jax.numpy as jnp
from jax import lax
from jax.experimental import pallas as pl
from jax.experimental.pallas import tpu as pltpu
from jax.experimental.pallas import tpu_sc as plsc

D = 32
NC, NS = 2, 16
NW = NC * NS
B = 16384 * 50
CHUNK = 128
ROWS_PER_W = B // (NW * CHUNK)
NBUF = 8
NGRP = ROWS_PER_W // NBUF


def _make_kernel():
    mesh = plsc.VectorSubcoreMesh(core_axis_name="c", subcore_axis_name="s")

    @functools.partial(
        pl.kernel,
        mesh=mesh,
        out_type=jax.ShapeDtypeStruct((B, D), jnp.float32),
        scratch_types=[
            pltpu.VMEM((ROWS_PER_W, CHUNK), jnp.int32),
            pltpu.VMEM((NBUF, CHUNK, D), jnp.float32),
            pltpu.SemaphoreType.DMA((NBUF,)),
            pltpu.SemaphoreType.DMA((NBUF,)),
        ],
        compiler_params=pltpu.CompilerParams(use_tc_tiling_on_sc=False),
    )
    def emb_kernel(idx_hbm, table_hbm, out_hbm, idx_v, rows_v, gsem, osem):
        wid = lax.axis_index("s") * NC + lax.axis_index("c")
        row0 = wid * ROWS_PER_W
        base = row0 * CHUNK
        pltpu.sync_copy(idx_hbm.at[pl.ds(row0, ROWS_PER_W)], idx_v)

        def start_gather(r, b):
            pltpu.async_copy(table_hbm.at[idx_v.at[r]], rows_v.at[b],
                             gsem.at[b])

        def wait_gather(r, b):
            pltpu.make_async_copy(table_hbm.at[idx_v.at[r]], rows_v.at[b],
                                  gsem.at[b]).wait()

        def start_out(r, b):
            pltpu.async_copy(rows_v.at[b],
                             out_hbm.at[pl.ds(base + r * CHUNK, CHUNK)],
                             osem.at[b])

        def wait_out(r, b):
            pltpu.make_async_copy(rows_v.at[b],
                                  out_hbm.at[pl.ds(base + r * CHUNK, CHUNK)],
                                  osem.at[b]).wait()

        for b in range(NBUF):
            start_gather(b, b)

        def grp_body(g, carry):
            for b in range(NBUF):
                r = g * NBUF + b
                wait_gather(r, b)
                start_out(r, b)
                wait_out(r, b)
                start_gather(r + NBUF, b)
            return carry

        lax.fori_loop(0, NGRP - 1, grp_body, 0)

        for b in range(NBUF):
            r = (NGRP - 1) * NBUF + b
            wait_gather(r, b)
            start_out(r, b)
            wait_out(r, b)

    return emb_kernel


_emb = _make_kernel()


@jax.jit
def kernel(token_ids, weights):
    idx = token_ids.reshape(B // CHUNK, CHUNK).astype(jnp.int32)
    out = _emb(idx, weights)
    return out.reshape(token_ids.shape[0], token_ids.shape[1], D)


# speedup vs baseline: 1.8050x; 1.6221x over previous
import functools

import jax
import jax.numpy as jnp
from jax import lax
from jax.experimental import pallas as pl
from jax.experimental.pallas import tpu as pltpu
from jax.experimental.pallas import tpu_sc as plsc

D = 32
T = 50
NI = 16384
NC, NS = 2, 16
NW = NC * NS
IPW = NI // NW
GB = 4
NSLOT = IPW // GB
NBUF = 8
NGRP = NSLOT // NBUF


def _make_kernel():
    mesh = plsc.VectorSubcoreMesh(core_axis_name="c", subcore_axis_name="s")

    @functools.partial(
        pl.kernel,
        mesh=mesh,
        out_type=jax.ShapeDtypeStruct((NI, T, D), jnp.float32),
        scratch_types=[
            pltpu.VMEM((IPW, T), jnp.int32),
            pltpu.VMEM((NBUF, GB, T, D), jnp.float32),
            pltpu.SemaphoreType.DMA((NBUF,)),
            pltpu.SemaphoreType.DMA((NBUF,)),
        ],
        compiler_params=pltpu.CompilerParams(use_tc_tiling_on_sc=False),
    )
    def emb_kernel(idx_hbm, table_hbm, out_hbm, idx_v, rows_v, gsem, osem):
        wid = lax.axis_index("s") * NC + lax.axis_index("c")
        row0 = wid * IPW
        pltpu.sync_copy(idx_hbm.at[pl.ds(row0, IPW)], idx_v)

        def start_slot(k, b):
            for j in range(GB):
                pltpu.async_copy(table_hbm.at[idx_v.at[k * GB + j]],
                                 rows_v.at[b, j], gsem.at[b])

        def wait_slot(k, b):
            for j in range(GB):
                pltpu.make_async_copy(table_hbm.at[idx_v.at[k * GB + j]],
                                      rows_v.at[b, j], gsem.at[b]).wait()

        def start_out(k, b):
            pltpu.async_copy(rows_v.at[b],
                             out_hbm.at[pl.ds(row0 + k * GB, GB)],
                             osem.at[b])

        def wait_out(k, b):
            pltpu.make_async_copy(rows_v.at[b],
                                  out_hbm.at[pl.ds(row0 + k * GB, GB)],
                                  osem.at[b]).wait()

        for b in range(NBUF):
            start_slot(b, b)

        def grp_body(g, carry):
            for b in range(NBUF):
                k = g * NBUF + b
                wait_slot(k, b)
                start_out(k, b)
                wait_out(k, b)
                start_slot(k + NBUF, b)
            return carry

        lax.fori_loop(0, NGRP - 1, grp_body, 0)

        for b in range(NBUF):
            k = (NGRP - 1) * NBUF + b
            wait_slot(k, b)
            start_out(k, b)
            wait_out(k, b)

    return emb_kernel


_emb = _make_kernel()


@jax.jit
def kernel(token_ids, weights):
    return _emb(token_ids, weights)
